# Initial kernel scaffold; baseline (speedup 1.0000x reference)
import jax
import jax.numpy as jnp
from jax.experimental import pallas as pl


def kernel(predictions, targets, bin_edges, bin_weights):
    raise NotImplementedError("write your pallas kernel here")


# speedup vs baseline: 1.5152x; 1.5152x over previous
import functools

import jax
import jax.numpy as jnp
from jax import lax
from jax.experimental import pallas as pl
from jax.experimental.pallas import tpu as pltpu
from jax.experimental.pallas import tpu_sc as plsc

_L = 16
_NW = 32
_UNROLL = 8


def _make_sc_loss(n_total: int, n_chunks: int):
    per_w = n_total // _NW
    chunk = per_w // n_chunks
    assert per_w % n_chunks == 0
    assert chunk % (_L * _UNROLL) == 0
    assert (per_w % 8) == 0 and (chunk % 8) == 0

    mesh = plsc.VectorSubcoreMesh(core_axis_name="c", subcore_axis_name="s")
    inv_n = 1.0 / float(n_total)

    @functools.partial(
        pl.kernel,
        mesh=mesh,
        out_type=jax.ShapeDtypeStruct((_NW * _L,), jnp.float32),
        scratch_types=[
            pltpu.VMEM((2, chunk), jnp.float32),
            pltpu.VMEM((2, chunk), jnp.float32),
            pltpu.VMEM((12, _L), jnp.float32),
            pltpu.VMEM((_L,), jnp.float32),
            pltpu.SemaphoreType.DMA,
            pltpu.SemaphoreType.DMA,
            pltpu.SemaphoreType.DMA,
            pltpu.SemaphoreType.DMA,
        ],
    )
    def sc_loss(p_hbm, t_hbm, par_hbm, out_hbm, pbuf, tbuf, par_v, out_v,
                sp0, sp1, st0, st1):
        cid = lax.axis_index("c")
        sid = lax.axis_index("s")
        wid = sid * 2 + cid
        base = wid * per_w

        psems = (sp0, sp1)
        tsems = (st0, st1)

        pltpu.sync_copy(par_hbm, par_v)
        edges = [par_v[j] for j in range(6)]
        wts = [par_v[6 + j] for j in range(6)]

        def start(k):
            slot = k % 2
            off = base + k * chunk
            cp = pltpu.async_copy(p_hbm.at[pl.ds(off, chunk)],
                                  pbuf.at[slot], psems[slot])
            ct = pltpu.async_copy(t_hbm.at[pl.ds(off, chunk)],
                                  tbuf.at[slot], tsems[slot])
            return cp, ct

        inflight = start(0)
        accs = tuple(jnp.zeros((_L,), jnp.float32) for _ in range(_UNROLL))

        for k in range(n_chunks):
            nxt = start(k + 1) if k + 1 < n_chunks else None
            inflight[0].wait()
            inflight[1].wait()
            slot = k % 2
            ps = pbuf.at[slot]
            ts = tbuf.at[slot]

            def body(i, a, ps=ps, ts=ts):
                o = i * (_L * _UNROLL)
                out = []
                for u in range(_UNROLL):
                    p = ps[pl.ds(o + u * _L, _L)]
                    t = ts[pl.ds(o + u * _L, _L)]
                    d = jnp.abs(p - t)
                    w = wts[5]
                    for j in range(6):
                        w = jnp.where(t > edges[j], wts[j], w)
                    out.append(a[u] + d * w)
                return tuple(out)

            accs = lax.fori_loop(0, chunk // (_L * _UNROLL), body, accs)
            inflight = nxt

        total = accs[0]
        for u in range(1, _UNROLL):
            total = total + accs[u]
        out_v[...] = total * inv_n
        pltpu.sync_copy(out_v, out_hbm.at[pl.ds(wid * _L, _L)])

    return sc_loss


def kernel(predictions, targets, bin_edges, bin_weights):
    n_total = predictions.size
    p = predictions.reshape(-1)
    t = targets.reshape(-1)
    params = jnp.concatenate(
        [jnp.broadcast_to(bin_edges[:6, None], (6, _L)),
         jnp.broadcast_to(bin_weights[:6, None], (6, _L))], axis=0)
    partials = _make_sc_loss(n_total, n_chunks=8)(p, t, params)
    return jnp.sum(partials)


# speedup vs baseline: 2.1569x; 1.4235x over previous
import functools

import jax
import jax.numpy as jnp
from jax import lax
from jax.experimental import pallas as pl
from jax.experimental.pallas import tpu as pltpu
from jax.experimental.pallas import tpu_sc as plsc

_L = 16
_NW = 32


def _make_sc_loss(n_rows: int, n_cols: int, n_chunks: int):
    rows_w = n_rows // _NW
    rows_c = rows_w // n_chunks
    assert rows_w % n_chunks == 0
    n_full = n_cols // _L
    tail = n_cols - n_full * _L
    inv_n = 1.0 / float(n_rows * n_cols)

    mesh = plsc.VectorSubcoreMesh(core_axis_name="c", subcore_axis_name="s")

    @functools.partial(
        pl.kernel,
        mesh=mesh,
        out_type=jax.ShapeDtypeStruct((_NW * _L,), jnp.float32),
        scratch_types=[
            pltpu.VMEM((2, rows_c, n_cols), jnp.float32),
            pltpu.VMEM((2, rows_c, n_cols), jnp.float32),
            pltpu.VMEM((12 * _L,), jnp.float32),
            pltpu.VMEM((_L,), jnp.float32),
            pltpu.SemaphoreType.DMA,
            pltpu.SemaphoreType.DMA,
            pltpu.SemaphoreType.DMA,
            pltpu.SemaphoreType.DMA,
        ],
    )
    def sc_loss(p_hbm, t_hbm, par_hbm, out_hbm, pbuf, tbuf, par_v, out_v,
                sp0, sp1, st0, st1):
        cid = lax.axis_index("c")
        sid = lax.axis_index("s")
        wid = sid * 2 + cid
        base = wid * rows_w

        psems = (sp0, sp1)
        tsems = (st0, st1)

        pltpu.sync_copy(par_hbm, par_v)
        edges = [par_v[pl.ds(j * _L, _L)] for j in range(6)]
        wts = [par_v[pl.ds((6 + j) * _L, _L)] for j in range(6)]
        lane = lax.iota(jnp.int32, _L)
        tail_w = jnp.where(lane >= (_L - tail), 1.0, 0.0).astype(jnp.float32)

        def start(k):
            slot = k % 2
            r0 = base + k * rows_c
            cp = pltpu.async_copy(p_hbm.at[pl.ds(r0, rows_c)],
                                  pbuf.at[slot], psems[slot])
            ct = pltpu.async_copy(t_hbm.at[pl.ds(r0, rows_c)],
                                  tbuf.at[slot], tsems[slot])
            return cp, ct

        def weighted(p, t):
            d = jnp.abs(p - t)
            w = wts[5]
            for j in range(6):
                w = jnp.where(t > edges[j], wts[j], w)
            return d * w

        inflight = start(0)
        accs = tuple(jnp.zeros((_L,), jnp.float32) for _ in range(n_full + 1))

        for k in range(n_chunks):
            nxt = start(k + 1) if k + 1 < n_chunks else None
            inflight[0].wait()
            inflight[1].wait()
            slot = k % 2
            ps = pbuf.at[slot]
            ts = tbuf.at[slot]

            def body(r, a, ps=ps, ts=ts):
                out = []
                for u in range(n_full):
                    p = ps[r, pl.ds(u * _L, _L)]
                    t = ts[r, pl.ds(u * _L, _L)]
                    out.append(a[u] + weighted(p, t))
                p = ps[r, pl.ds(n_cols - _L, _L)]
                t = ts[r, pl.ds(n_cols - _L, _L)]
                out.append(a[n_full] + weighted(p, t) * tail_w)
                return tuple(out)

            accs = lax.fori_loop(0, rows_c, body, accs)
            inflight = nxt

        total = accs[0]
        for u in range(1, n_full + 1):
            total = total + accs[u]
        out_v[...] = total * inv_n
        pltpu.sync_copy(out_v, out_hbm.at[pl.ds(wid * _L, _L)])

    return sc_loss


def kernel(predictions, targets, bin_edges, bin_weights):
    n_rows, n_cols = predictions.shape
    params = jnp.concatenate(
        [jnp.broadcast_to(bin_edges[:6, None], (6, _L)),
         jnp.broadcast_to(bin_weights[:6, None], (6, _L))],
        axis=0).reshape(-1)
    partials = _make_sc_loss(n_rows, n_cols, n_chunks=8)(
        predictions, targets, params)
    return jnp.sum(partials)
